# baseline (speedup 1.0000x reference)
import jax
import jax.numpy as jnp
from jax.experimental import pallas as pl
from jax.experimental.pallas import tpu as pltpu

BLOCK_ROWS = 2048


def _copy_body(x_hbm, o_hbm, sizes_ref, buf, insem, outsem):
    i = pl.program_id(0)
    nblocks = pl.num_programs(0)

    @pl.when(i == 0)
    def _():
        sizes_ref[0] = jnp.int32(nblocks * BLOCK_ROWS)

    def in_copy(g, slot):
        return pltpu.make_async_copy(
            x_hbm.at[pl.ds(g * BLOCK_ROWS, BLOCK_ROWS)],
            buf.at[slot],
            insem.at[slot],
        )

    def out_copy(g, slot):
        return pltpu.make_async_copy(
            buf.at[slot],
            o_hbm.at[pl.ds(g * BLOCK_ROWS, BLOCK_ROWS)],
            outsem.at[slot],
        )

    slot = i % 2

    @pl.when(i == 0)
    def _():
        in_copy(0, 0).start()

    @pl.when(i + 1 < nblocks)
    def _():
        next_slot = (i + 1) % 2

        @pl.when(i >= 1)
        def _():
            out_copy(i - 1, next_slot).wait()

        in_copy(i + 1, next_slot).start()

    in_copy(i, slot).wait()
    out_copy(i, slot).start()

    @pl.when(i == nblocks - 1)
    def _():
        @pl.when(nblocks >= 2)
        def _():
            out_copy(i - 1, (i - 1) % 2).wait()

        out_copy(i, slot).wait()


def kernel(x):
    n, d = x.shape
    gathered, sizes = pl.pallas_call(
        _copy_body,
        grid=(n // BLOCK_ROWS,),
        in_specs=[pl.BlockSpec(memory_space=pl.ANY)],
        out_specs=[
            pl.BlockSpec(memory_space=pl.ANY),
            pl.BlockSpec(memory_space=pltpu.MemorySpace.SMEM),
        ],
        out_shape=[
            jax.ShapeDtypeStruct((n, d), x.dtype),
            jax.ShapeDtypeStruct((1,), jnp.int32),
        ],
        scratch_shapes=[
            pltpu.VMEM((2, BLOCK_ROWS, d), x.dtype),
            pltpu.SemaphoreType.DMA((2,)),
            pltpu.SemaphoreType.DMA((2,)),
        ],
    )(x)
    return (gathered, sizes)


# speedup vs baseline: 1.0237x; 1.0237x over previous
import jax
import jax.numpy as jnp
from jax.experimental import pallas as pl
from jax.experimental.pallas import tpu as pltpu

BLOCK_ROWS = 2048


def _copy_body(x_ref, o_ref, sizes_ref):
    @pl.when(pl.program_id(0) == 0)
    def _():
        sizes_ref[0] = jnp.int32(pl.num_programs(0) * BLOCK_ROWS)

    o_ref[...] = x_ref[...]


def kernel(x):
    n, d = x.shape
    gathered, sizes = pl.pallas_call(
        _copy_body,
        grid=(n // BLOCK_ROWS,),
        in_specs=[pl.BlockSpec((BLOCK_ROWS, d), lambda i: (i, 0))],
        out_specs=[
            pl.BlockSpec((BLOCK_ROWS, d), lambda i: (i, 0)),
            pl.BlockSpec(memory_space=pltpu.MemorySpace.SMEM),
        ],
        out_shape=[
            jax.ShapeDtypeStruct((n, d), x.dtype),
            jax.ShapeDtypeStruct((1,), jnp.int32),
        ],
    )(x)
    return (gathered, sizes)
